# baseline (speedup 1.0000x reference)
import jax
import jax.numpy as jnp
from jax.experimental import pallas as pl
from jax.experimental.pallas import tpu as pltpu

_B, _NS, _NO = 4, 10475, 2048
_THRESHOLD = 0.1
_BIG = 1e30
_BS = 256
_NSP = ((_NS + _BS - 1) // _BS) * _BS
_NCHUNK = _NSP // _BS


def _cl_kernel(s_ref, o_ref, loss_ref, valid_ref):
    ox = o_ref[0, 0:1, :]
    oy = o_ref[0, 1:2, :]
    oz = o_ref[0, 2:3, :]
    om = o_ref[0, 3:4, :]
    o2 = ox * ox + oy * oy + oz * oz
    oxm2 = -2.0 * ox.astype(jnp.bfloat16).astype(jnp.float32)
    oym2 = -2.0 * oy.astype(jnp.bfloat16).astype(jnp.float32)
    ozm2 = -2.0 * oz.astype(jnp.bfloat16).astype(jnp.float32)
    omask = om > _THRESHOLD
    om2m = jnp.where(omask, om * om, _BIG)
    no = jnp.sum(omask.astype(jnp.float32))

    def body(i, carry):
        hsum, nsum, oacc = carry
        sc = s_ref[0, pl.ds(i * _BS, _BS), :]
        sx = sc[:, 0:1]
        sy = sc[:, 1:2]
        sz = sc[:, 2:3]
        sm = sc[:, 3:4]
        s2 = sx * sx + sy * sy + sz * sz
        smask = sm > _THRESHOLD
        sm2m = jnp.where(smask, sm * sm, _BIG)
        smw = jnp.where(smask, sm, 0.0)
        sxb = sx.astype(jnp.bfloat16).astype(jnp.float32)
        syb = sy.astype(jnp.bfloat16).astype(jnp.float32)
        szb = sz.astype(jnp.bfloat16).astype(jnp.float32)
        t = (((s2 + o2) + sxb * oxm2) + syb * oym2) + szb * ozm2
        d2 = jnp.maximum(t, 1e-12)
        v1 = d2 * om2m
        rmin = jnp.min(v1, axis=1, keepdims=True)
        hsum = hsum + smw * jnp.sqrt(rmin)
        nsum = nsum + smask.astype(jnp.float32)
        v2 = d2 * sm2m
        oacc = jnp.minimum(oacc, jnp.min(v2, axis=0, keepdims=True))
        return hsum, nsum, oacc

    init = (jnp.zeros((_BS, 1), jnp.float32), jnp.zeros((_BS, 1), jnp.float32),
            jnp.full((1, _NO), _BIG, jnp.float32))
    hvec, nvec, oacc = jax.lax.fori_loop(0, _NCHUNK, body, init)
    hsum = jnp.sum(hvec)
    ns = jnp.sum(nvec)

    osum = jnp.sum(jnp.where(omask, om * jnp.sqrt(oacc), 0.0))
    h2o_mean = hsum / jnp.maximum(ns, 1.0)
    o2h_mean = osum / jnp.maximum(no, 1.0)
    valid = jnp.logical_and(ns > 0, no > 0)
    contrib = jnp.where(valid, h2o_mean + o2h_mean, 0.0)

    loss_ref[...] = contrib.reshape(1, 1, 1)
    valid_ref[...] = valid.astype(jnp.float32).reshape(1, 1, 1)


def kernel(smplx_v, object_v, smpl_occlusion_maps, object_occlusion_maps,
           smpl_mean_occlusion_map, object_mean_occlusion_map):
    sm = smpl_occlusion_maps * smpl_mean_occlusion_map[None, :]
    om = object_occlusion_maps * object_mean_occlusion_map[None, :]

    s_all = jnp.concatenate([smplx_v, sm[:, :, None]], axis=2)
    s_all = jnp.pad(s_all, ((0, 0), (0, _NSP - _NS), (0, 0)))

    o_all = jnp.concatenate(
        [object_v.transpose(0, 2, 1), om[:, None, :]], axis=1)

    loss, valid = pl.pallas_call(
        _cl_kernel,
        grid=(_B,),
        in_specs=[
            pl.BlockSpec((1, _NSP, 4), lambda b: (b, 0, 0)),
            pl.BlockSpec((1, 4, _NO), lambda b: (b, 0, 0)),
        ],
        out_specs=[
            pl.BlockSpec((1, 1, 1), lambda b: (b, 0, 0)),
            pl.BlockSpec((1, 1, 1), lambda b: (b, 0, 0)),
        ],
        out_shape=[
            jax.ShapeDtypeStruct((_B, 1, 1), jnp.float32),
            jax.ShapeDtypeStruct((_B, 1, 1), jnp.float32),
        ],
        compiler_params=pltpu.CompilerParams(
            dimension_semantics=("parallel",)),
    )(s_all, o_all)

    total = jnp.sum(loss)
    count = jnp.sum(valid)
    return jnp.where(count > 0, total / jnp.maximum(count, 1.0), total)


# speedup vs baseline: 1.2354x; 1.2354x over previous
import dataclasses

import jax
import jax.numpy as jnp
from jax.experimental import pallas as pl
from jax.experimental.pallas import tpu as pltpu
from jax.experimental.pallas import tpu_sc as plsc

_B, _NS, _NO = 4, 10475, 2048
_THRESHOLD = 0.1
_BIG = 1e30
_BS = 512
_NSP = 11264
_NCHUNK = _NSP // _BS


def _compact_kernel(s4t_hbm, z_hbm, out_hbm, cnt_hbm, inbuf, outbuf, cntbuf,
                    sem):
    c = jax.lax.axis_index("core")
    s = jax.lax.axis_index("subcore")
    tile = c * 16 + s

    @pl.when(tile < _B)
    def _():
        b = tile
        pltpu.async_copy(s4t_hbm.at[b], inbuf, sem).wait()
        pltpu.async_copy(z_hbm, outbuf, sem).wait()

        def chunk(i, cnt):
            w = inbuf[pl.ds(3 * _NSP + i * 16, 16)]
            mask = w > _THRESHOLD
            mi = mask.astype(jnp.int32)
            pos = cnt + plsc.cumsum(mi) - 1
            pos = jnp.where(mask, pos, 0)
            base4 = pos * 4
            for coord in range(4):
                plsc.store_scatter(
                    outbuf, [base4 + coord],
                    inbuf[pl.ds(coord * _NSP + i * 16, 16)], mask=mask)
            return cnt + jnp.sum(mi)

        cnt = jax.lax.fori_loop(0, _NSP // 16, chunk, jnp.int32(0))
        cntbuf[...] = jnp.full((16,), cnt, jnp.int32)
        pltpu.async_copy(outbuf, out_hbm.at[b], sem).wait()
        pltpu.async_copy(cntbuf, cnt_hbm.at[b], sem).wait()


def _sc_compact(s4t, zeros4):
    mesh = plsc.VectorSubcoreMesh(core_axis_name="core",
                                  subcore_axis_name="subcore")
    cp = pltpu.CompilerParams()
    if "needs_layout_passes" in pltpu.CompilerParams.__dataclass_fields__:
        cp = dataclasses.replace(cp, needs_layout_passes=False)
    return pl.kernel(
        _compact_kernel,
        out_type=[
            jax.ShapeDtypeStruct((_B, _NSP * 4), jnp.float32),
            jax.ShapeDtypeStruct((_B, 16), jnp.int32),
        ],
        mesh=mesh,
        scratch_types=[
            pltpu.VMEM((4 * _NSP,), jnp.float32),
            pltpu.VMEM((_NSP * 4,), jnp.float32),
            pltpu.VMEM((16,), jnp.int32),
            pltpu.SemaphoreType.DMA,
        ],
        compiler_params=cp,
    )(s4t, zeros4)


def _cl_kernel(cnt_ref, s_ref, o_ref, loss_ref, valid_ref):
    b = pl.program_id(0)
    cnt = cnt_ref[b, 0]

    ox = o_ref[0, 0:1, :]
    oy = o_ref[0, 1:2, :]
    oz = o_ref[0, 2:3, :]
    om = o_ref[0, 3:4, :]
    o2 = ox * ox + oy * oy + oz * oz
    oxm2 = -2.0 * ox.astype(jnp.bfloat16).astype(jnp.float32)
    oym2 = -2.0 * oy.astype(jnp.bfloat16).astype(jnp.float32)
    ozm2 = -2.0 * oz.astype(jnp.bfloat16).astype(jnp.float32)
    omask = om > _THRESHOLD
    om2m = jnp.where(omask, om * om, _BIG)
    no = jnp.sum(omask.astype(jnp.float32))

    def body(i, carry):
        hsum, oacc = carry
        base = i * _BS
        sc = s_ref[0, pl.ds(base, _BS), :]
        sx = sc[:, 0:1]
        sy = sc[:, 1:2]
        sz = sc[:, 2:3]
        sm = sc[:, 3:4]
        s2 = sx * sx + sy * sy + sz * sz
        rows = jax.lax.broadcasted_iota(jnp.int32, (_BS, 1), 0) + base
        valid = rows < cnt
        sm2m = jnp.where(valid, sm * sm, _BIG)
        smw = jnp.where(valid, sm, 0.0)
        sxb = sx.astype(jnp.bfloat16).astype(jnp.float32)
        syb = sy.astype(jnp.bfloat16).astype(jnp.float32)
        szb = sz.astype(jnp.bfloat16).astype(jnp.float32)
        t = (((s2 + o2) + sxb * oxm2) + syb * oym2) + szb * ozm2
        d2 = jnp.maximum(t, 1e-12)
        v1 = d2 * om2m
        rmin = jnp.min(v1, axis=1, keepdims=True)
        hsum = hsum + smw * jnp.sqrt(rmin)
        v2 = d2 * sm2m
        oacc = jnp.minimum(oacc, jnp.min(v2, axis=0, keepdims=True))
        return hsum, oacc

    nch = (cnt + _BS - 1) // _BS
    init = (jnp.zeros((_BS, 1), jnp.float32),
            jnp.full((1, _NO), _BIG, jnp.float32))
    hvec, oacc = jax.lax.fori_loop(0, nch, body, init)
    hsum = jnp.sum(hvec)
    ns = cnt.astype(jnp.float32)

    osum = jnp.sum(jnp.where(omask, om * jnp.sqrt(oacc), 0.0))
    h2o_mean = hsum / jnp.maximum(ns, 1.0)
    o2h_mean = osum / jnp.maximum(no, 1.0)
    valid_b = jnp.logical_and(ns > 0, no > 0)
    contrib = jnp.where(valid_b, h2o_mean + o2h_mean, 0.0)

    loss_ref[...] = contrib.reshape(1, 1, 1)
    valid_ref[...] = valid_b.astype(jnp.float32).reshape(1, 1, 1)


def kernel(smplx_v, object_v, smpl_occlusion_maps, object_occlusion_maps,
           smpl_mean_occlusion_map, object_mean_occlusion_map):
    sm = smpl_occlusion_maps * smpl_mean_occlusion_map[None, :]
    om = object_occlusion_maps * object_mean_occlusion_map[None, :]

    s_all = jnp.concatenate([smplx_v, sm[:, :, None]], axis=2)
    s_all = jnp.pad(s_all, ((0, 0), (0, _NSP - _NS), (0, 0)))
    s4t = s_all.transpose(0, 2, 1).reshape(_B, 4 * _NSP)
    zeros4 = jnp.zeros((_NSP * 4,), jnp.float32)

    s_comp, counts = _sc_compact(s4t, zeros4)
    s_comp = s_comp.reshape(_B, _NSP, 4)

    o_all = jnp.concatenate(
        [object_v.transpose(0, 2, 1), om[:, None, :]], axis=1)

    loss, valid = pl.pallas_call(
        _cl_kernel,
        grid=(_B,),
        in_specs=[
            pl.BlockSpec(memory_space=pltpu.SMEM),
            pl.BlockSpec((1, _NSP, 4), lambda b: (b, 0, 0)),
            pl.BlockSpec((1, 4, _NO), lambda b: (b, 0, 0)),
        ],
        out_specs=[
            pl.BlockSpec((1, 1, 1), lambda b: (b, 0, 0)),
            pl.BlockSpec((1, 1, 1), lambda b: (b, 0, 0)),
        ],
        out_shape=[
            jax.ShapeDtypeStruct((_B, 1, 1), jnp.float32),
            jax.ShapeDtypeStruct((_B, 1, 1), jnp.float32),
        ],
    )(counts, s_comp, o_all)

    total = jnp.sum(loss)
    count = jnp.sum(valid)
    return jnp.where(count > 0, total / jnp.maximum(count, 1.0), total)
